# Initial kernel scaffold; baseline (speedup 1.0000x reference)
import jax
import jax.numpy as jnp
from jax.experimental import pallas as pl


def kernel(items, plastic_weights, reward, W1, b1, W2, b2, W3, b3, Wc, bc, Wr, br, Wn, bn, alpha, Wv, bv):
    raise NotImplementedError("write your pallas kernel here")


# speedup vs baseline: 1.1675x; 1.1675x over previous
import jax
import jax.numpy as jnp
from jax.experimental import pallas as pl
from jax.experimental.pallas import tpu as pltpu

_B, _IN, _H = 128, 512, 1024


def _prologue_body(items_ref, reward_ref, W1_ref, b1_ref, W2_ref, b2_ref,
                   Wr_ref, br_ref, emb_ref, cur_ref, rew_ref):
    nt = (((1,), (1,)), ((), ()))
    emb = jnp.tanh(
        jax.lax.dot_general(items_ref[...], W1_ref[...], nt,
                            preferred_element_type=jnp.float32) + b1_ref[...])
    emb_ref[...] = emb
    cur_ref[...] = jax.lax.dot_general(
        emb, W2_ref[...], nt, preferred_element_type=jnp.float32) + b2_ref[...]
    rew_ref[...] = reward_ref[...] * Wr_ref[...] + br_ref[...]


def _main_body(plastic_ref, emb_ref, cur_ref, rew_ref, alpha_ref, W3_ref,
               b3_ref, Wn_ref, bn_ref, Wc_ref, bc_ref, Wv_ref, bv_ref,
               choice_ref, nm_ref, value_ref, newp_ref, hid_ref):
    nt = (((1,), (1,)), ((), ()))
    P = plastic_ref[0]
    emb = emb_ref[0]
    M = alpha_ref[...] * P
    contrib = jax.lax.dot_general(emb, M, nt,
                                  preferred_element_type=jnp.float32)
    pre = cur_ref[0] + contrib
    hidden = jnp.tanh(pre)
    hid_ref[0] = hidden
    t = hidden + rew_ref[0]
    h3 = jnp.tanh(
        jax.lax.dot_general(t, W3_ref[...], nt,
                            preferred_element_type=jnp.float32) + b3_ref[...])
    nmo = jnp.tanh(
        jax.lax.dot_general(h3, Wn_ref[...], nt,
                            preferred_element_type=jnp.float32) + bn_ref[...])
    nm = nmo[:, 0:1] - nmo[:, 1:2]
    nm_ref[0] = nm
    choice_ref[0] = jax.nn.sigmoid(
        jnp.sum(hidden * Wc_ref[...], axis=1, keepdims=True) + bc_ref[...])
    value_ref[0] = jnp.sum(hidden * Wv_ref[...], axis=1, keepdims=True) + bv_ref[...]
    outer = jax.lax.dot_general(pre, emb, (((0,), (0,)), ((), ())),
                                preferred_element_type=jnp.float32)
    newp_ref[0] = jnp.clip(P + nm * (jnp.tanh(outer) * 10.0), -50.0, 50.0)


def kernel(items, plastic_weights, reward, W1, b1, W2, b2, W3, b3,
           Wc, bc, Wr, br, Wn, bn, alpha, Wv, bv):
    f32 = jnp.float32
    B, IN, H = _B, _IN, _H
    half = B // 2

    emb, cur, rew = pl.pallas_call(
        _prologue_body,
        grid=(2,),
        in_specs=[
            pl.BlockSpec((half, IN), lambda i: (i, 0)),
            pl.BlockSpec((half, 1), lambda i: (i, 0)),
            pl.BlockSpec((H, IN), lambda i: (0, 0)),
            pl.BlockSpec((1, H), lambda i: (0, 0)),
            pl.BlockSpec((H, H), lambda i: (0, 0)),
            pl.BlockSpec((1, H), lambda i: (0, 0)),
            pl.BlockSpec((1, H), lambda i: (0, 0)),
            pl.BlockSpec((1, H), lambda i: (0, 0)),
        ],
        out_specs=[
            pl.BlockSpec((half, H), lambda i: (i, 0)),
            pl.BlockSpec((half, H), lambda i: (i, 0)),
            pl.BlockSpec((half, H), lambda i: (i, 0)),
        ],
        out_shape=[jax.ShapeDtypeStruct((B, H), f32)] * 3,
        compiler_params=pltpu.CompilerParams(
            dimension_semantics=("parallel",)),
    )(items, reward, W1, b1.reshape(1, H), W2, b2.reshape(1, H),
      Wr.reshape(1, H), br.reshape(1, H))

    emb3 = emb.reshape(B, 1, H)
    cur3 = cur.reshape(B, 1, H)
    rew3 = rew.reshape(B, 1, H)

    row3 = pl.BlockSpec((1, 1, H), lambda b: (b, 0, 0))
    full = lambda shape: pl.BlockSpec(shape, lambda b: tuple(0 for _ in shape))

    choice3, nm3, value3, newp, hid3 = pl.pallas_call(
        _main_body,
        grid=(B,),
        in_specs=[
            pl.BlockSpec((1, H, H), lambda b: (b, 0, 0)),
            row3, row3, row3,
            full((H, H)),
            full((H, H)),
            full((1, H)),
            full((2, H)),
            full((1, 2)),
            full((1, H)),
            full((1, 1)),
            full((1, H)),
            full((1, 1)),
        ],
        out_specs=[
            pl.BlockSpec((1, 1, 1), lambda b: (b, 0, 0)),
            pl.BlockSpec((1, 1, 1), lambda b: (b, 0, 0)),
            pl.BlockSpec((1, 1, 1), lambda b: (b, 0, 0)),
            pl.BlockSpec((1, H, H), lambda b: (b, 0, 0)),
            row3,
        ],
        out_shape=[
            jax.ShapeDtypeStruct((B, 1, 1), f32),
            jax.ShapeDtypeStruct((B, 1, 1), f32),
            jax.ShapeDtypeStruct((B, 1, 1), f32),
            jax.ShapeDtypeStruct((B, H, H), f32),
            jax.ShapeDtypeStruct((B, 1, H), f32),
        ],
        compiler_params=pltpu.CompilerParams(
            dimension_semantics=("parallel",),
            vmem_limit_bytes=100 * 1024 * 1024),
    )(plastic_weights, emb3, cur3, rew3, alpha, W3, b3.reshape(1, H),
      Wn, bn.reshape(1, 2), Wc, bc.reshape(1, 1), Wv, bv.reshape(1, 1))

    return (choice3.reshape(B, 1), nm3, value3.reshape(B, 1),
            newp, hid3.reshape(B, H))


# speedup vs baseline: 1.3853x; 1.1865x over previous
import jax
import jax.numpy as jnp
from jax.experimental import pallas as pl
from jax.experimental.pallas import tpu as pltpu

_B, _IN, _H = 128, 512, 1024
_G = 2

_NT = (((1,), (1,)), ((), ()))
_OUTER = (((0,), (0,)), ((), ()))


def _prologue_body(items_ref, reward_ref, W1_ref, b1_ref, W2_ref, b2_ref,
                   Wr_ref, br_ref, emb_ref, cur_ref, rew_ref):
    emb = jnp.tanh(
        jax.lax.dot_general(items_ref[...], W1_ref[...], _NT,
                            preferred_element_type=jnp.float32) + b1_ref[...])
    emb_ref[...] = emb
    cur_ref[...] = jax.lax.dot_general(
        emb, W2_ref[...], _NT, preferred_element_type=jnp.float32) + b2_ref[...]
    rew_ref[...] = reward_ref[...] * Wr_ref[...] + br_ref[...]


def _main_body(plastic_ref, emb_ref, cur_ref, rew_ref, alpha_ref, W3_ref,
               b3_ref, Wn_ref, bn_ref, Wc_ref, bc_ref, Wv_ref, bv_ref,
               choice_ref, nm_ref, value_ref, newp_ref, hid_ref):
    G, H = _G, _H
    alpha = alpha_ref[...]
    emb = emb_ref[...].reshape(G, H)
    pres = []
    for g in range(G):
        M = alpha * plastic_ref[g]
        contrib = jax.lax.dot_general(emb[g:g + 1], M, _NT,
                                      preferred_element_type=jnp.float32)
        pres.append(contrib)
    pre = jnp.concatenate(pres, axis=0) + cur_ref[...].reshape(G, H)
    hidden = jnp.tanh(pre)
    hid_ref[...] = hidden.reshape(G, 1, H)
    t = hidden + rew_ref[...].reshape(G, H)
    h3 = jnp.tanh(
        jax.lax.dot_general(t, W3_ref[...], _NT,
                            preferred_element_type=jnp.float32) + b3_ref[...])
    nmo = jnp.tanh(
        jax.lax.dot_general(h3, Wn_ref[...], _NT,
                            preferred_element_type=jnp.float32) + bn_ref[...])
    nm = nmo[:, 0:1] - nmo[:, 1:2]
    nm_ref[...] = nm.reshape(G, 1, 1)
    choice_ref[...] = jax.nn.sigmoid(
        jnp.sum(hidden * Wc_ref[...], axis=1, keepdims=True)
        + bc_ref[...]).reshape(G, 1, 1)
    value_ref[...] = (jnp.sum(hidden * Wv_ref[...], axis=1, keepdims=True)
                      + bv_ref[...]).reshape(G, 1, 1)
    nm10 = nm * 10.0
    for g in range(G):
        outer = jax.lax.dot_general(pre[g:g + 1], emb[g:g + 1], _OUTER,
                                    preferred_element_type=jnp.float32)
        newp_ref[g] = jnp.clip(
            plastic_ref[g] + nm10[g:g + 1] * jnp.tanh(outer), -50.0, 50.0)


def kernel(items, plastic_weights, reward, W1, b1, W2, b2, W3, b3,
           Wc, bc, Wr, br, Wn, bn, alpha, Wv, bv):
    f32 = jnp.float32
    B, IN, H, G = _B, _IN, _H, _G
    half = B // 2

    emb, cur, rew = pl.pallas_call(
        _prologue_body,
        grid=(2,),
        in_specs=[
            pl.BlockSpec((half, IN), lambda i: (i, 0)),
            pl.BlockSpec((half, 1), lambda i: (i, 0)),
            pl.BlockSpec((H, IN), lambda i: (0, 0)),
            pl.BlockSpec((1, H), lambda i: (0, 0)),
            pl.BlockSpec((H, H), lambda i: (0, 0)),
            pl.BlockSpec((1, H), lambda i: (0, 0)),
            pl.BlockSpec((1, H), lambda i: (0, 0)),
            pl.BlockSpec((1, H), lambda i: (0, 0)),
        ],
        out_specs=[
            pl.BlockSpec((half, H), lambda i: (i, 0)),
            pl.BlockSpec((half, H), lambda i: (i, 0)),
            pl.BlockSpec((half, H), lambda i: (i, 0)),
        ],
        out_shape=[jax.ShapeDtypeStruct((B, H), f32)] * 3,
        compiler_params=pltpu.CompilerParams(
            dimension_semantics=("parallel",)),
    )(items, reward, W1, b1.reshape(1, H), W2, b2.reshape(1, H),
      Wr.reshape(1, H), br.reshape(1, H))

    emb3 = emb.reshape(B, 1, H)
    cur3 = cur.reshape(B, 1, H)
    rew3 = rew.reshape(B, 1, H)

    row3 = pl.BlockSpec((G, 1, H), lambda b: (b, 0, 0))
    full = lambda shape: pl.BlockSpec(shape, lambda b: tuple(0 for _ in shape))

    choice3, nm3, value3, newp, hid3 = pl.pallas_call(
        _main_body,
        grid=(B // G,),
        in_specs=[
            pl.BlockSpec((G, H, H), lambda b: (b, 0, 0)),
            row3, row3, row3,
            full((H, H)),
            full((H, H)),
            full((1, H)),
            full((2, H)),
            full((1, 2)),
            full((1, H)),
            full((1, 1)),
            full((1, H)),
            full((1, 1)),
        ],
        out_specs=[
            pl.BlockSpec((G, 1, 1), lambda b: (b, 0, 0)),
            pl.BlockSpec((G, 1, 1), lambda b: (b, 0, 0)),
            pl.BlockSpec((G, 1, 1), lambda b: (b, 0, 0)),
            pl.BlockSpec((G, H, H), lambda b: (b, 0, 0)),
            row3,
        ],
        out_shape=[
            jax.ShapeDtypeStruct((B, 1, 1), f32),
            jax.ShapeDtypeStruct((B, 1, 1), f32),
            jax.ShapeDtypeStruct((B, 1, 1), f32),
            jax.ShapeDtypeStruct((B, H, H), f32),
            jax.ShapeDtypeStruct((B, 1, H), f32),
        ],
        compiler_params=pltpu.CompilerParams(
            dimension_semantics=("parallel",),
            vmem_limit_bytes=100 * 1024 * 1024),
    )(plastic_weights, emb3, cur3, rew3, alpha, W3, b3.reshape(1, H),
      Wn, bn.reshape(1, 2), Wc, bc.reshape(1, 1), Wv, bv.reshape(1, 1))

    return (choice3.reshape(B, 1), nm3, value3.reshape(B, 1),
            newp, hid3.reshape(B, H))
